# Initial kernel scaffold; baseline (speedup 1.0000x reference)
import functools

import jax
import jax.numpy as jnp
from jax import lax
from jax.experimental import pallas as pl
from jax.experimental.pallas import tpu as pltpu
from jax.experimental.pallas import tpu_sc as plsc

GRID = 160
LANES = 16
SUB = 128
GROUPS = SUB // LANES
CHUNK = 2048
NW = 32

_OFFS = (0, 1, GRID, GRID + 1,
         GRID * GRID, GRID * GRID + 1, GRID * GRID + GRID, GRID * GRID + GRID + 1)


def _sc_body(n_pts, pw_hbm, ph_hbm, pd_hbm, dens_hbm, params_hbm, out_hbm,
             pw_v, ph_v, pd_v, idx_v, gat_v, w_v, out_v, params_v, sem):
    wid = lax.axis_index("s") * 2 + lax.axis_index("c")
    per_w = n_pts // NW
    base_w = wid * per_w

    pltpu.sync_copy(params_hbm, params_v)
    off_w = params_v[0]
    scl_w = params_v[1]
    off_h = params_v[2]
    scl_h = params_v[3]
    off_d = params_v[4]
    scl_d = params_v[5]
    thr = params_v[6]

    def axis_prep(vals, off, scl):
        s = jnp.maximum((vals - off) * scl, 0.0)
        i0 = jnp.minimum(s.astype(jnp.int32), GRID - 2)
        frac = s - i0.astype(jnp.float32)
        return i0, frac

    def chunk_body(ci, carry):
        base = base_w + ci * CHUNK
        pltpu.sync_copy(pw_hbm.at[pl.ds(base, CHUNK)], pw_v)
        pltpu.sync_copy(ph_hbm.at[pl.ds(base, CHUNK)], ph_v)
        pltpu.sync_copy(pd_hbm.at[pl.ds(base, CHUNK)], pd_v)

        def sub_body(si, carry2):
            s0 = si * SUB
            for g in range(GROUPS):
                o = s0 + g * LANES
                x0, fx = axis_prep(pw_v[pl.ds(o, LANES)], off_w, scl_w)
                y0, fy = axis_prep(ph_v[pl.ds(o, LANES)], off_h, scl_h)
                z0, fz = axis_prep(pd_v[pl.ds(o, LANES)], off_d, scl_d)
                b = (z0 * GRID + y0) * GRID + x0
                gl = pl.ds(g * LANES, LANES)
                for k in range(8):
                    idx_v[k, gl] = b + _OFFS[k]
                w_v[0, gl] = fx
                w_v[1, gl] = fy
                w_v[2, gl] = fz
            copies = [pltpu.async_copy(dens_hbm.at[idx_v.at[k]], gat_v.at[k], sem)
                      for k in range(8)]
            for cp in copies:
                cp.wait()
            for g in range(GROUPS):
                gl = pl.ds(g * LANES, LANES)
                fx = w_v[0, gl]
                fy = w_v[1, gl]
                fz = w_v[2, gl]
                gx = 1.0 - fx
                gy = 1.0 - fy
                gz = 1.0 - fz
                c00 = gat_v[0, gl] * gx + gat_v[1, gl] * fx
                c01 = gat_v[2, gl] * gx + gat_v[3, gl] * fx
                c10 = gat_v[4, gl] * gx + gat_v[5, gl] * fx
                c11 = gat_v[6, gl] * gx + gat_v[7, gl] * fx
                c0 = c00 * gy + c01 * fy
                c1 = c10 * gy + c11 * fy
                d = c0 * gz + c1 * fz
                out_v[pl.ds(s0 + g * LANES, LANES)] = jnp.where(
                    d >= thr, 1.0, 0.0)
            return carry2

        lax.fori_loop(0, CHUNK // SUB, sub_body, 0)
        pltpu.sync_copy(out_v, out_hbm.at[pl.ds(base, CHUNK)])
        return carry

    lax.fori_loop(0, per_w // CHUNK, chunk_body, 0)


@functools.partial(jax.jit, static_argnums=(5,))
def _gridmask_sc(pw, ph, pd, dens_flat, params16, n_pts):
    kern = pl.kernel(
        functools.partial(_sc_body, n_pts),
        out_type=jax.ShapeDtypeStruct((n_pts,), jnp.float32),
        mesh=plsc.VectorSubcoreMesh(core_axis_name="c", subcore_axis_name="s"),
        scratch_types=[
            pltpu.VMEM((CHUNK,), jnp.float32),
            pltpu.VMEM((CHUNK,), jnp.float32),
            pltpu.VMEM((CHUNK,), jnp.float32),
            pltpu.VMEM((8, SUB), jnp.int32),
            pltpu.VMEM((8, SUB), jnp.float32),
            pltpu.VMEM((3, SUB), jnp.float32),
            pltpu.VMEM((CHUNK,), jnp.float32),
            pltpu.VMEM((8, LANES), jnp.float32),
            pltpu.SemaphoreType.DMA,
        ],
    )
    return kern(pw, ph, pd, dens_flat, params16)


def kernel(xyz, density, xyz_min, xyz_max, act_shift, voxel_size_ratio,
           mask_cache_thres):
    shape = xyz.shape[:-1]
    pts = xyz.reshape(-1, 3)
    n_pts = pts.shape[0]
    pw = pts[:, 2]
    ph = pts[:, 1]
    pd = pts[:, 0]
    dens_flat = density.reshape(-1)

    c = -jnp.log1p(-mask_cache_thres) / voxel_size_ratio
    thr = jnp.log(jnp.expm1(c)) - act_shift

    scl = (GRID - 1.0) / (xyz_max - xyz_min)
    params = jnp.stack([
        xyz_min[2], scl[2],
        xyz_min[1], scl[1],
        xyz_min[0], scl[0],
        thr, jnp.float32(0.0),
    ]).astype(jnp.float32)
    params16 = jnp.broadcast_to(params[:, None], (8, LANES))

    outf = _gridmask_sc(pw, ph, pd, dens_flat, params16, n_pts)
    return (outf > 0.5).reshape(shape)


# speedup vs baseline: 6.0104x; 6.0104x over previous
import functools

import jax
import jax.numpy as jnp
from jax import lax
from jax.experimental import pallas as pl
from jax.experimental.pallas import tpu as pltpu
from jax.experimental.pallas import tpu_sc as plsc

GRID = 160
LANES = 16
SUB = 128
GROUPS = SUB // LANES
CHUNK = 2048
NW = 32

_OFFS = (0, 1, GRID, GRID + 1,
         GRID * GRID, GRID * GRID + 1, GRID * GRID + GRID, GRID * GRID + GRID + 1)


def _sc_body(n_pts, pw_hbm, ph_hbm, pd_hbm, dens_hbm, params_hbm, out_hbm,
             pw_v, ph_v, pd_v, idx_v, gat_v, w_v, out_v, params_v, sem):
    wid = lax.axis_index("s") * 2 + lax.axis_index("c")
    per_w = n_pts // NW
    base_w = wid * per_w

    pltpu.sync_copy(params_hbm, params_v)
    off_w = params_v[0]
    scl_w = params_v[1]
    off_h = params_v[2]
    scl_h = params_v[3]
    off_d = params_v[4]
    scl_d = params_v[5]
    thr = params_v[6]

    def axis_prep(vals, off, scl):
        s = jnp.maximum((vals - off) * scl, 0.0)
        i0 = jnp.minimum(s.astype(jnp.int32), GRID - 2)
        frac = s - i0.astype(jnp.float32)
        return i0, frac

    def chunk_body(ci, carry):
        base = base_w + ci * CHUNK
        pltpu.sync_copy(pw_hbm.at[pl.ds(base, CHUNK)], pw_v)
        pltpu.sync_copy(ph_hbm.at[pl.ds(base, CHUNK)], ph_v)
        pltpu.sync_copy(pd_hbm.at[pl.ds(base, CHUNK)], pd_v)

        def sub_body(si, carry2):
            s0 = si * SUB
            for g in range(GROUPS):
                o = s0 + g * LANES
                x0, fx = axis_prep(pw_v[pl.ds(o, LANES)], off_w, scl_w)
                y0, fy = axis_prep(ph_v[pl.ds(o, LANES)], off_h, scl_h)
                z0, fz = axis_prep(pd_v[pl.ds(o, LANES)], off_d, scl_d)
                b = (z0 * GRID + y0) * GRID + x0
                gl = pl.ds(g * LANES, LANES)
                for k in range(8):
                    idx_v[k, gl] = b + _OFFS[k]
                w_v[0, gl] = fx
                w_v[1, gl] = fy
                w_v[2, gl] = fz
            copies = [pltpu.async_copy(dens_hbm.at[idx_v.at[k]], gat_v.at[k], sem)
                      for k in range(8)]
            for cp in copies:
                cp.wait()
            for g in range(GROUPS):
                gl = pl.ds(g * LANES, LANES)
                fx = w_v[0, gl]
                fy = w_v[1, gl]
                fz = w_v[2, gl]
                gx = 1.0 - fx
                gy = 1.0 - fy
                gz = 1.0 - fz
                c00 = gat_v[0, gl] * gx + gat_v[1, gl] * fx
                c01 = gat_v[2, gl] * gx + gat_v[3, gl] * fx
                c10 = gat_v[4, gl] * gx + gat_v[5, gl] * fx
                c11 = gat_v[6, gl] * gx + gat_v[7, gl] * fx
                c0 = c00 * gy + c01 * fy
                c1 = c10 * gy + c11 * fy
                d = c0 * gz + c1 * fz
                out_v[pl.ds(s0 + g * LANES, LANES)] = jnp.where(
                    d >= thr, 1.0, 0.0)
            return carry2

        lax.fori_loop(0, CHUNK // SUB, sub_body, 0)
        pltpu.sync_copy(out_v, out_hbm.at[pl.ds(base, CHUNK)])
        return carry

    lax.fori_loop(0, per_w // CHUNK, chunk_body, 0)


@functools.partial(jax.jit, static_argnums=(5,))
def _gridmask_sc(pw, ph, pd, dens_flat, params16, n_pts):
    kern = pl.kernel(
        functools.partial(_sc_body, n_pts),
        out_type=jax.ShapeDtypeStruct((n_pts,), jnp.float32),
        mesh=plsc.VectorSubcoreMesh(core_axis_name="c", subcore_axis_name="s",
                                    num_cores=2, num_subcores=16),
        scratch_types=[
            pltpu.VMEM((CHUNK,), jnp.float32),
            pltpu.VMEM((CHUNK,), jnp.float32),
            pltpu.VMEM((CHUNK,), jnp.float32),
            pltpu.VMEM((8, SUB), jnp.int32),
            pltpu.VMEM((8, SUB), jnp.float32),
            pltpu.VMEM((3, SUB), jnp.float32),
            pltpu.VMEM((CHUNK,), jnp.float32),
            pltpu.VMEM((8, LANES), jnp.float32),
            pltpu.SemaphoreType.DMA,
        ],
    )
    return kern(pw, ph, pd, dens_flat, params16)


def kernel(xyz, density, xyz_min, xyz_max, act_shift, voxel_size_ratio,
           mask_cache_thres):
    shape = xyz.shape[:-1]
    pts = xyz.reshape(-1, 3)
    n_pts = pts.shape[0]
    pw = pts[:, 2]
    ph = pts[:, 1]
    pd = pts[:, 0]
    dens_flat = density.reshape(-1)

    c = -jnp.log1p(-mask_cache_thres) / voxel_size_ratio
    thr = jnp.log(jnp.expm1(c)) - act_shift

    scl = (GRID - 1.0) / (xyz_max - xyz_min)
    params = jnp.stack([
        xyz_min[2], scl[2],
        xyz_min[1], scl[1],
        xyz_min[0], scl[0],
        thr, jnp.float32(0.0),
    ]).astype(jnp.float32)
    params16 = jnp.broadcast_to(params[:, None], (8, LANES))

    outf = _gridmask_sc(pw, ph, pd, dens_flat, params16, n_pts)
    return (outf > 0.5).reshape(shape)


# speedup vs baseline: 7.4524x; 1.2399x over previous
import functools

import jax
import jax.numpy as jnp
from jax import lax
from jax.experimental import pallas as pl
from jax.experimental.pallas import tpu as pltpu
from jax.experimental.pallas import tpu_sc as plsc

GRID = 160
LANES = 16
SUB = 128
GROUPS = SUB // LANES
CHUNK = 2048
NW = 32

_OFFS = (0, 1, GRID, GRID + 1,
         GRID * GRID, GRID * GRID + 1, GRID * GRID + GRID, GRID * GRID + GRID + 1)


def _sc_body(n_pts, pw_hbm, ph_hbm, pd_hbm, dens_hbm, params_hbm, out_hbm,
             pw_v, ph_v, pd_v, idx_v, gat_v, w_v, out_v, params_v, sem):
    wid = lax.axis_index("s") * 2 + lax.axis_index("c")
    per_w = n_pts // NW
    base_w = wid * per_w
    n_sub = CHUNK // SUB

    pltpu.sync_copy(params_hbm, params_v)
    off_w = params_v[0]
    scl_w = params_v[1]
    off_h = params_v[2]
    scl_h = params_v[3]
    off_d = params_v[4]
    scl_d = params_v[5]
    thr = params_v[6]

    def axis_prep(vals, off, scl):
        s = jnp.maximum((vals - off) * scl, 0.0)
        i0 = jnp.minimum(s.astype(jnp.int32), GRID - 2)
        frac = s - i0.astype(jnp.float32)
        return i0, frac

    def compute_sub(si, p):
        s0 = si * SUB
        for g in range(GROUPS):
            o = s0 + g * LANES
            x0, fx = axis_prep(pw_v[pl.ds(o, LANES)], off_w, scl_w)
            y0, fy = axis_prep(ph_v[pl.ds(o, LANES)], off_h, scl_h)
            z0, fz = axis_prep(pd_v[pl.ds(o, LANES)], off_d, scl_d)
            b = (z0 * GRID + y0) * GRID + x0
            gl = pl.ds(g * LANES, LANES)
            for k in range(8):
                idx_v[p, k, gl] = b + _OFFS[k]
            w_v[p, 0, gl] = fx
            w_v[p, 1, gl] = fy
            w_v[p, 2, gl] = fz

    def fire(p):
        for k in range(8):
            pltpu.async_copy(dens_hbm.at[idx_v.at[p, k]], gat_v.at[p, k], sem)

    def drain(p):
        for k in range(8):
            pltpu.make_async_copy(dens_hbm.at[idx_v.at[p, k]],
                                  gat_v.at[p, k], sem).wait()

    def blend_sub(si, p):
        s0 = si * SUB
        for g in range(GROUPS):
            gl = pl.ds(g * LANES, LANES)
            fx = w_v[p, 0, gl]
            fy = w_v[p, 1, gl]
            fz = w_v[p, 2, gl]
            gx = 1.0 - fx
            gy = 1.0 - fy
            gz = 1.0 - fz
            c00 = gat_v[p, 0, gl] * gx + gat_v[p, 1, gl] * fx
            c01 = gat_v[p, 2, gl] * gx + gat_v[p, 3, gl] * fx
            c10 = gat_v[p, 4, gl] * gx + gat_v[p, 5, gl] * fx
            c11 = gat_v[p, 6, gl] * gx + gat_v[p, 7, gl] * fx
            c0 = c00 * gy + c01 * fy
            c1 = c10 * gy + c11 * fy
            d = c0 * gz + c1 * fz
            out_v[pl.ds(s0 + g * LANES, LANES)] = jnp.where(d >= thr, 1.0, 0.0)

    def chunk_body(ci, carry):
        base = base_w + ci * CHUNK
        pltpu.sync_copy(pw_hbm.at[pl.ds(base, CHUNK)], pw_v)
        pltpu.sync_copy(ph_hbm.at[pl.ds(base, CHUNK)], ph_v)
        pltpu.sync_copy(pd_hbm.at[pl.ds(base, CHUNK)], pd_v)

        compute_sub(0, 0)
        fire(0)

        def pipe_body(si, carry2):
            p = si % 2
            q = 1 - p
            compute_sub(si, p)
            fire(p)
            drain(q)
            blend_sub(si - 1, q)
            return carry2

        lax.fori_loop(1, n_sub, pipe_body, 0)
        pl_last = (n_sub - 1) % 2
        drain(pl_last)
        blend_sub(n_sub - 1, pl_last)
        pltpu.sync_copy(out_v, out_hbm.at[pl.ds(base, CHUNK)])
        return carry

    lax.fori_loop(0, per_w // CHUNK, chunk_body, 0)


@functools.partial(jax.jit, static_argnums=(5,))
def _gridmask_sc(pw, ph, pd, dens_flat, params16, n_pts):
    kern = pl.kernel(
        functools.partial(_sc_body, n_pts),
        out_type=jax.ShapeDtypeStruct((n_pts,), jnp.float32),
        mesh=plsc.VectorSubcoreMesh(core_axis_name="c", subcore_axis_name="s",
                                    num_cores=2, num_subcores=16),
        scratch_types=[
            pltpu.VMEM((CHUNK,), jnp.float32),
            pltpu.VMEM((CHUNK,), jnp.float32),
            pltpu.VMEM((CHUNK,), jnp.float32),
            pltpu.VMEM((2, 8, SUB), jnp.int32),
            pltpu.VMEM((2, 8, SUB), jnp.float32),
            pltpu.VMEM((2, 3, SUB), jnp.float32),
            pltpu.VMEM((CHUNK,), jnp.float32),
            pltpu.VMEM((8, LANES), jnp.float32),
            pltpu.SemaphoreType.DMA,
        ],
    )
    return kern(pw, ph, pd, dens_flat, params16)


def kernel(xyz, density, xyz_min, xyz_max, act_shift, voxel_size_ratio,
           mask_cache_thres):
    shape = xyz.shape[:-1]
    pts = xyz.reshape(-1, 3)
    n_pts = pts.shape[0]
    pw = pts[:, 2]
    ph = pts[:, 1]
    pd = pts[:, 0]
    dens_flat = density.reshape(-1)

    c = -jnp.log1p(-mask_cache_thres) / voxel_size_ratio
    thr = jnp.log(jnp.expm1(c)) - act_shift

    scl = (GRID - 1.0) / (xyz_max - xyz_min)
    params = jnp.stack([
        xyz_min[2], scl[2],
        xyz_min[1], scl[1],
        xyz_min[0], scl[0],
        thr, jnp.float32(0.0),
    ]).astype(jnp.float32)
    params16 = jnp.broadcast_to(params[:, None], (8, LANES))

    outf = _gridmask_sc(pw, ph, pd, dens_flat, params16, n_pts)
    return (outf > 0.5).reshape(shape)


# speedup vs baseline: 7.4863x; 1.0045x over previous
import functools

import jax
import jax.numpy as jnp
from jax import lax
from jax.experimental import pallas as pl
from jax.experimental.pallas import tpu as pltpu
from jax.experimental.pallas import tpu_sc as plsc

GRID = 160
LANES = 16
SUB = 128
GROUPS = SUB // LANES
CHUNK = 2048
NW = 32

_OFFS = (0, 1, GRID, GRID + 1,
         GRID * GRID, GRID * GRID + 1, GRID * GRID + GRID, GRID * GRID + GRID + 1)


def _sc_body(n_pts, pw_hbm, ph_hbm, pd_hbm, dens_hbm, params_hbm, out_hbm,
             pw_v, ph_v, pd_v, idx_a, idx_b, gat_a, gat_b, w_v, out_v,
             params_v, sem):
    wid = lax.axis_index("s") * 2 + lax.axis_index("c")
    per_w = n_pts // NW
    base_w = wid * per_w
    n_sub = CHUNK // SUB

    pltpu.sync_copy(params_hbm, params_v)
    off_w = params_v[0]
    scl_w = params_v[1]
    off_h = params_v[2]
    scl_h = params_v[3]
    off_d = params_v[4]
    scl_d = params_v[5]
    thr = params_v[6]

    def axis_prep(vals, off, scl):
        s = jnp.maximum((vals - off) * scl, 0.0)
        i0 = jnp.minimum(s.astype(jnp.int32), GRID - 2)
        frac = s - i0.astype(jnp.float32)
        return i0, frac

    def compute_sub(si, idx_v, p):
        s0 = si * SUB
        for g in range(GROUPS):
            o = s0 + g * LANES
            x0, fx = axis_prep(pw_v[pl.ds(o, LANES)], off_w, scl_w)
            y0, fy = axis_prep(ph_v[pl.ds(o, LANES)], off_h, scl_h)
            z0, fz = axis_prep(pd_v[pl.ds(o, LANES)], off_d, scl_d)
            b = (z0 * GRID + y0) * GRID + x0
            gl = pl.ds(g * LANES, LANES)
            for k in range(8):
                idx_v[pl.ds(k * SUB + g * LANES, LANES)] = b + _OFFS[k]
            w_v[p, 0, gl] = fx
            w_v[p, 1, gl] = fy
            w_v[p, 2, gl] = fz

    def fire(idx_v, gat_v):
        pltpu.async_copy(dens_hbm.at[idx_v], gat_v, sem)

    def drain(idx_v, gat_v):
        pltpu.make_async_copy(dens_hbm.at[idx_v], gat_v, sem).wait()

    def blend_sub(si, gat_v, p):
        s0 = si * SUB
        for g in range(GROUPS):
            gl = pl.ds(g * LANES, LANES)
            fx = w_v[p, 0, gl]
            fy = w_v[p, 1, gl]
            fz = w_v[p, 2, gl]
            gx = 1.0 - fx
            gy = 1.0 - fy
            gz = 1.0 - fz

            def corner(k):
                return gat_v[pl.ds(k * SUB + g * LANES, LANES)]

            c00 = corner(0) * gx + corner(1) * fx
            c01 = corner(2) * gx + corner(3) * fx
            c10 = corner(4) * gx + corner(5) * fx
            c11 = corner(6) * gx + corner(7) * fx
            c0 = c00 * gy + c01 * fy
            c1 = c10 * gy + c11 * fy
            d = c0 * gz + c1 * fz
            out_v[pl.ds(s0 + g * LANES, LANES)] = jnp.where(d >= thr, 1.0, 0.0)

    def chunk_body(ci, carry):
        base = base_w + ci * CHUNK
        pltpu.sync_copy(pw_hbm.at[pl.ds(base, CHUNK)], pw_v)
        pltpu.sync_copy(ph_hbm.at[pl.ds(base, CHUNK)], ph_v)
        pltpu.sync_copy(pd_hbm.at[pl.ds(base, CHUNK)], pd_v)

        compute_sub(0, idx_a, 0)
        fire(idx_a, gat_a)

        def pipe_body(h, carry2):
            si = 2 * h + 1
            compute_sub(si, idx_b, 1)
            fire(idx_b, gat_b)
            drain(idx_a, gat_a)
            blend_sub(si - 1, gat_a, 0)
            compute_sub(si + 1, idx_a, 0)
            fire(idx_a, gat_a)
            drain(idx_b, gat_b)
            blend_sub(si, gat_b, 1)
            return carry2

        lax.fori_loop(0, n_sub // 2 - 1, pipe_body, 0)
        si = n_sub - 1
        compute_sub(si, idx_b, 1)
        fire(idx_b, gat_b)
        drain(idx_a, gat_a)
        blend_sub(si - 1, gat_a, 0)
        drain(idx_b, gat_b)
        blend_sub(si, gat_b, 1)
        pltpu.sync_copy(out_v, out_hbm.at[pl.ds(base, CHUNK)])
        return carry

    lax.fori_loop(0, per_w // CHUNK, chunk_body, 0)


@functools.partial(jax.jit, static_argnums=(5,))
def _gridmask_sc(pw, ph, pd, dens_flat, params16, n_pts):
    kern = pl.kernel(
        functools.partial(_sc_body, n_pts),
        out_type=jax.ShapeDtypeStruct((n_pts,), jnp.float32),
        mesh=plsc.VectorSubcoreMesh(core_axis_name="c", subcore_axis_name="s",
                                    num_cores=2, num_subcores=16),
        scratch_types=[
            pltpu.VMEM((CHUNK,), jnp.float32),
            pltpu.VMEM((CHUNK,), jnp.float32),
            pltpu.VMEM((CHUNK,), jnp.float32),
            pltpu.VMEM((8 * SUB,), jnp.int32),
            pltpu.VMEM((8 * SUB,), jnp.int32),
            pltpu.VMEM((8 * SUB,), jnp.float32),
            pltpu.VMEM((8 * SUB,), jnp.float32),
            pltpu.VMEM((2, 3, SUB), jnp.float32),
            pltpu.VMEM((CHUNK,), jnp.float32),
            pltpu.VMEM((8, LANES), jnp.float32),
            pltpu.SemaphoreType.DMA,
        ],
    )
    return kern(pw, ph, pd, dens_flat, params16)


def kernel(xyz, density, xyz_min, xyz_max, act_shift, voxel_size_ratio,
           mask_cache_thres):
    shape = xyz.shape[:-1]
    pts = xyz.reshape(-1, 3)
    n_pts = pts.shape[0]
    pw = pts[:, 2]
    ph = pts[:, 1]
    pd = pts[:, 0]
    dens_flat = density.reshape(-1)

    c = -jnp.log1p(-mask_cache_thres) / voxel_size_ratio
    thr = jnp.log(jnp.expm1(c)) - act_shift

    scl = (GRID - 1.0) / (xyz_max - xyz_min)
    params = jnp.stack([
        xyz_min[2], scl[2],
        xyz_min[1], scl[1],
        xyz_min[0], scl[0],
        thr, jnp.float32(0.0),
    ]).astype(jnp.float32)
    params16 = jnp.broadcast_to(params[:, None], (8, LANES))

    outf = _gridmask_sc(pw, ph, pd, dens_flat, params16, n_pts)
    return (outf > 0.5).reshape(shape)


# speedup vs baseline: 8.3210x; 1.1115x over previous
import functools

import jax
import jax.numpy as jnp
from jax import lax
from jax.experimental import pallas as pl
from jax.experimental.pallas import tpu as pltpu
from jax.experimental.pallas import tpu_sc as plsc

GRID = 160
LANES = 16
SUB = 128
GROUPS = SUB // LANES
CHUNK = 2048
NW = 32

_OFFS = (0, 1, GRID, GRID + 1,
         GRID * GRID, GRID * GRID + 1, GRID * GRID + GRID, GRID * GRID + GRID + 1)


def _sc_body(n_pts, pw_hbm, ph_hbm, pd_hbm, dens_hbm, params_hbm, out_hbm,
             pw_v, ph_v, pd_v, idx_a, idx_b, idx_c, idx_d,
             gat_a, gat_b, gat_c, gat_d, w_v, out_v, params_v, sem):
    wid = lax.axis_index("s") * 2 + lax.axis_index("c")
    per_w = n_pts // NW
    base_w = wid * per_w
    n_sub = CHUNK // SUB

    pltpu.sync_copy(params_hbm, params_v)
    off_w = params_v[0]
    scl_w = params_v[1]
    off_h = params_v[2]
    scl_h = params_v[3]
    off_d = params_v[4]
    scl_d = params_v[5]
    thr = params_v[6]

    def axis_prep(vals, off, scl):
        s = jnp.maximum((vals - off) * scl, 0.0)
        i0 = jnp.minimum(s.astype(jnp.int32), GRID - 2)
        frac = s - i0.astype(jnp.float32)
        return i0, frac

    def compute_sub(si, idx_v, p):
        s0 = si * SUB
        for g in range(GROUPS):
            o = s0 + g * LANES
            x0, fx = axis_prep(pw_v[pl.ds(o, LANES)], off_w, scl_w)
            y0, fy = axis_prep(ph_v[pl.ds(o, LANES)], off_h, scl_h)
            z0, fz = axis_prep(pd_v[pl.ds(o, LANES)], off_d, scl_d)
            b = (z0 * GRID + y0) * GRID + x0
            gl = pl.ds(g * LANES, LANES)
            for k in range(8):
                idx_v[pl.ds(g * 8 * LANES + k * LANES, LANES)] = b + _OFFS[k]
            w_v[p, 0, gl] = fx
            w_v[p, 1, gl] = fy
            w_v[p, 2, gl] = fz

    def fire(idx_v, gat_v):
        pltpu.async_copy(dens_hbm.at[idx_v], gat_v, sem)

    def drain(idx_v, gat_v):
        pltpu.make_async_copy(dens_hbm.at[idx_v], gat_v, sem).wait()

    def blend_sub(si, gat_v, p):
        s0 = si * SUB
        for g in range(GROUPS):
            gl = pl.ds(g * LANES, LANES)
            fx = w_v[p, 0, gl]
            fy = w_v[p, 1, gl]
            fz = w_v[p, 2, gl]
            gx = 1.0 - fx
            gy = 1.0 - fy
            gz = 1.0 - fz

            def corner(k):
                return gat_v[pl.ds(g * 8 * LANES + k * LANES, LANES)]

            c00 = corner(0) * gx + corner(1) * fx
            c01 = corner(2) * gx + corner(3) * fx
            c10 = corner(4) * gx + corner(5) * fx
            c11 = corner(6) * gx + corner(7) * fx
            c0 = c00 * gy + c01 * fy
            c1 = c10 * gy + c11 * fy
            d = c0 * gz + c1 * fz
            out_v[pl.ds(s0 + g * LANES, LANES)] = jnp.where(d >= thr, 1.0, 0.0)

    def chunk_body(ci, carry):
        base = base_w + ci * CHUNK
        pltpu.sync_copy(pw_hbm.at[pl.ds(base, CHUNK)], pw_v)
        pltpu.sync_copy(ph_hbm.at[pl.ds(base, CHUNK)], ph_v)
        pltpu.sync_copy(pd_hbm.at[pl.ds(base, CHUNK)], pd_v)

        bufs = ((idx_a, gat_a), (idx_b, gat_b), (idx_c, gat_c), (idx_d, gat_d))
        for si in range(3):
            compute_sub(si, bufs[si][0], si)
            fire(*bufs[si])

        def pipe_body(h, carry2):
            s0 = 4 * h + 3
            for j in range(4):
                si = s0 + j
                cb = (3 + j) % 4
                compute_sub(si, bufs[cb][0], cb)
                fire(*bufs[cb])
                drain(*bufs[j])
                blend_sub(si - 3, bufs[j][1], j)
            return carry2

        lax.fori_loop(0, (n_sub - 4) // 4, pipe_body, 0)
        si = n_sub - 1
        compute_sub(si, bufs[3][0], 3)
        fire(*bufs[3])
        for j in range(4):
            drain(*bufs[j])
            blend_sub(si - 3 + j, bufs[j][1], j)
        pltpu.sync_copy(out_v, out_hbm.at[pl.ds(base, CHUNK)])
        return carry

    lax.fori_loop(0, per_w // CHUNK, chunk_body, 0)


@functools.partial(jax.jit, static_argnums=(5,))
def _gridmask_sc(pw, ph, pd, dens_flat, params16, n_pts):
    kern = pl.kernel(
        functools.partial(_sc_body, n_pts),
        out_type=jax.ShapeDtypeStruct((n_pts,), jnp.float32),
        mesh=plsc.VectorSubcoreMesh(core_axis_name="c", subcore_axis_name="s",
                                    num_cores=2, num_subcores=16),
        scratch_types=[
            pltpu.VMEM((CHUNK,), jnp.float32),
            pltpu.VMEM((CHUNK,), jnp.float32),
            pltpu.VMEM((CHUNK,), jnp.float32),
            pltpu.VMEM((8 * SUB,), jnp.int32),
            pltpu.VMEM((8 * SUB,), jnp.int32),
            pltpu.VMEM((8 * SUB,), jnp.int32),
            pltpu.VMEM((8 * SUB,), jnp.int32),
            pltpu.VMEM((8 * SUB,), jnp.float32),
            pltpu.VMEM((8 * SUB,), jnp.float32),
            pltpu.VMEM((8 * SUB,), jnp.float32),
            pltpu.VMEM((8 * SUB,), jnp.float32),
            pltpu.VMEM((4, 3, SUB), jnp.float32),
            pltpu.VMEM((CHUNK,), jnp.float32),
            pltpu.VMEM((8, LANES), jnp.float32),
            pltpu.SemaphoreType.DMA,
        ],
    )
    return kern(pw, ph, pd, dens_flat, params16)


def kernel(xyz, density, xyz_min, xyz_max, act_shift, voxel_size_ratio,
           mask_cache_thres):
    shape = xyz.shape[:-1]
    pts = xyz.reshape(-1, 3)
    n_pts = pts.shape[0]
    pw = pts[:, 2]
    ph = pts[:, 1]
    pd = pts[:, 0]
    dens_flat = density.reshape(-1)

    c = -jnp.log1p(-mask_cache_thres) / voxel_size_ratio
    thr = jnp.log(jnp.expm1(c)) - act_shift

    scl = (GRID - 1.0) / (xyz_max - xyz_min)
    params = jnp.stack([
        xyz_min[2], scl[2],
        xyz_min[1], scl[1],
        xyz_min[0], scl[0],
        thr, jnp.float32(0.0),
    ]).astype(jnp.float32)
    params16 = jnp.broadcast_to(params[:, None], (8, LANES))

    outf = _gridmask_sc(pw, ph, pd, dens_flat, params16, n_pts)
    return (outf > 0.5).reshape(shape)


# speedup vs baseline: 8.7187x; 1.0478x over previous
import functools

import jax
import jax.numpy as jnp
from jax import lax
from jax.experimental import pallas as pl
from jax.experimental.pallas import tpu as pltpu
from jax.experimental.pallas import tpu_sc as plsc

GRID = 160
LANES = 16
SUB = 256
GROUPS = SUB // LANES
CHUNK = 4096
NW = 32

_OFFS = (0, 1, GRID, GRID + 1,
         GRID * GRID, GRID * GRID + 1, GRID * GRID + GRID, GRID * GRID + GRID + 1)


def _sc_body(n_pts, pw_hbm, ph_hbm, pd_hbm, dens_hbm, params_hbm, out_hbm,
             pw_v, ph_v, pd_v, idx_a, idx_b, idx_c, idx_d,
             gat_a, gat_b, gat_c, gat_d, w_v, out_v, params_v, sem):
    wid = lax.axis_index("s") * 2 + lax.axis_index("c")
    per_w = n_pts // NW
    base_w = wid * per_w
    n_sub = CHUNK // SUB

    pltpu.sync_copy(params_hbm, params_v)
    off_w = params_v[0]
    scl_w = params_v[1]
    off_h = params_v[2]
    scl_h = params_v[3]
    off_d = params_v[4]
    scl_d = params_v[5]
    thr = params_v[6]

    def axis_prep(vals, off, scl):
        s = jnp.maximum((vals - off) * scl, 0.0)
        i0 = jnp.minimum(s.astype(jnp.int32), GRID - 2)
        frac = s - i0.astype(jnp.float32)
        return i0, frac

    def compute_sub(si, idx_v, p):
        s0 = si * SUB
        for g in range(GROUPS):
            o = s0 + g * LANES
            x0, fx = axis_prep(pw_v[pl.ds(o, LANES)], off_w, scl_w)
            y0, fy = axis_prep(ph_v[pl.ds(o, LANES)], off_h, scl_h)
            z0, fz = axis_prep(pd_v[pl.ds(o, LANES)], off_d, scl_d)
            b = (z0 * GRID + y0) * GRID + x0
            gl = pl.ds(g * LANES, LANES)
            for k in range(8):
                idx_v[pl.ds(g * 8 * LANES + k * LANES, LANES)] = b + _OFFS[k]
            w_v[p, 0, gl] = fx
            w_v[p, 1, gl] = fy
            w_v[p, 2, gl] = fz

    def fire(idx_v, gat_v):
        pltpu.async_copy(dens_hbm.at[idx_v], gat_v, sem)

    def drain(idx_v, gat_v):
        pltpu.make_async_copy(dens_hbm.at[idx_v], gat_v, sem).wait()

    def blend_sub(si, gat_v, p):
        s0 = si * SUB
        for g in range(GROUPS):
            gl = pl.ds(g * LANES, LANES)
            fx = w_v[p, 0, gl]
            fy = w_v[p, 1, gl]
            fz = w_v[p, 2, gl]
            gx = 1.0 - fx
            gy = 1.0 - fy
            gz = 1.0 - fz

            def corner(k):
                return gat_v[pl.ds(g * 8 * LANES + k * LANES, LANES)]

            c00 = corner(0) * gx + corner(1) * fx
            c01 = corner(2) * gx + corner(3) * fx
            c10 = corner(4) * gx + corner(5) * fx
            c11 = corner(6) * gx + corner(7) * fx
            c0 = c00 * gy + c01 * fy
            c1 = c10 * gy + c11 * fy
            d = c0 * gz + c1 * fz
            out_v[pl.ds(s0 + g * LANES, LANES)] = jnp.where(d >= thr, 1.0, 0.0)

    def chunk_body(ci, carry):
        base = base_w + ci * CHUNK
        pltpu.sync_copy(pw_hbm.at[pl.ds(base, CHUNK)], pw_v)
        pltpu.sync_copy(ph_hbm.at[pl.ds(base, CHUNK)], ph_v)
        pltpu.sync_copy(pd_hbm.at[pl.ds(base, CHUNK)], pd_v)

        bufs = ((idx_a, gat_a), (idx_b, gat_b), (idx_c, gat_c), (idx_d, gat_d))
        for si in range(3):
            compute_sub(si, bufs[si][0], si)
            fire(*bufs[si])

        def pipe_body(h, carry2):
            s0 = 4 * h + 3
            for j in range(4):
                si = s0 + j
                cb = (3 + j) % 4
                compute_sub(si, bufs[cb][0], cb)
                fire(*bufs[cb])
                drain(*bufs[j])
                blend_sub(si - 3, bufs[j][1], j)
            return carry2

        lax.fori_loop(0, (n_sub - 4) // 4, pipe_body, 0)
        si = n_sub - 1
        compute_sub(si, bufs[3][0], 3)
        fire(*bufs[3])
        for j in range(4):
            drain(*bufs[j])
            blend_sub(si - 3 + j, bufs[j][1], j)
        pltpu.sync_copy(out_v, out_hbm.at[pl.ds(base, CHUNK)])
        return carry

    lax.fori_loop(0, per_w // CHUNK, chunk_body, 0)


@functools.partial(jax.jit, static_argnums=(5,))
def _gridmask_sc(pw, ph, pd, dens_flat, params16, n_pts):
    kern = pl.kernel(
        functools.partial(_sc_body, n_pts),
        out_type=jax.ShapeDtypeStruct((n_pts,), jnp.float32),
        mesh=plsc.VectorSubcoreMesh(core_axis_name="c", subcore_axis_name="s",
                                    num_cores=2, num_subcores=16),
        scratch_types=[
            pltpu.VMEM((CHUNK,), jnp.float32),
            pltpu.VMEM((CHUNK,), jnp.float32),
            pltpu.VMEM((CHUNK,), jnp.float32),
            pltpu.VMEM((8 * SUB,), jnp.int32),
            pltpu.VMEM((8 * SUB,), jnp.int32),
            pltpu.VMEM((8 * SUB,), jnp.int32),
            pltpu.VMEM((8 * SUB,), jnp.int32),
            pltpu.VMEM((8 * SUB,), jnp.float32),
            pltpu.VMEM((8 * SUB,), jnp.float32),
            pltpu.VMEM((8 * SUB,), jnp.float32),
            pltpu.VMEM((8 * SUB,), jnp.float32),
            pltpu.VMEM((4, 3, SUB), jnp.float32),
            pltpu.VMEM((CHUNK,), jnp.float32),
            pltpu.VMEM((8, LANES), jnp.float32),
            pltpu.SemaphoreType.DMA,
        ],
    )
    return kern(pw, ph, pd, dens_flat, params16)


def kernel(xyz, density, xyz_min, xyz_max, act_shift, voxel_size_ratio,
           mask_cache_thres):
    shape = xyz.shape[:-1]
    pts = xyz.reshape(-1, 3)
    n_pts = pts.shape[0]
    pw = pts[:, 2]
    ph = pts[:, 1]
    pd = pts[:, 0]
    dens_flat = density.reshape(-1)

    c = -jnp.log1p(-mask_cache_thres) / voxel_size_ratio
    thr = jnp.log(jnp.expm1(c)) - act_shift

    scl = (GRID - 1.0) / (xyz_max - xyz_min)
    params = jnp.stack([
        xyz_min[2], scl[2],
        xyz_min[1], scl[1],
        xyz_min[0], scl[0],
        thr, jnp.float32(0.0),
    ]).astype(jnp.float32)
    params16 = jnp.broadcast_to(params[:, None], (8, LANES))

    outf = _gridmask_sc(pw, ph, pd, dens_flat, params16, n_pts)
    return (outf > 0.5).reshape(shape)


# speedup vs baseline: 8.7464x; 1.0032x over previous
import functools

import jax
import jax.numpy as jnp
from jax import lax
from jax.experimental import pallas as pl
from jax.experimental.pallas import tpu as pltpu
from jax.experimental.pallas import tpu_sc as plsc

GRID = 160
LANES = 16
SUB = 256
GROUPS = SUB // LANES
CHUNK = 4096
NW = 32

_OFFS = (0, 1, GRID, GRID + 1,
         GRID * GRID, GRID * GRID + 1, GRID * GRID + GRID, GRID * GRID + GRID + 1)


def _sc_body(n_pts, pw_hbm, ph_hbm, pd_hbm, dens_hbm, params_hbm, out_hbm,
             pw_v, ph_v, pd_v, idx_a, idx_b, idx_c, idx_d,
             gat_a, gat_b, gat_c, gat_d, out_v, params_v, sem):
    wid = lax.axis_index("s") * 2 + lax.axis_index("c")
    per_w = n_pts // NW
    base_w = wid * per_w
    n_sub = CHUNK // SUB

    pltpu.sync_copy(params_hbm, params_v)
    off_w = params_v[0]
    scl_w = params_v[1]
    off_h = params_v[2]
    scl_h = params_v[3]
    off_d = params_v[4]
    scl_d = params_v[5]
    thr = params_v[6]

    def axis_prep(vals, off, scl):
        s = jnp.maximum((vals - off) * scl, 0.0)
        i0 = jnp.minimum(s.astype(jnp.int32), GRID - 2)
        frac = s - i0.astype(jnp.float32)
        return i0, frac

    def compute_sub(si, idx_v, p):
        s0 = si * SUB
        for g in range(GROUPS):
            o = s0 + g * LANES
            x0, _ = axis_prep(pw_v[pl.ds(o, LANES)], off_w, scl_w)
            y0, _ = axis_prep(ph_v[pl.ds(o, LANES)], off_h, scl_h)
            z0, _ = axis_prep(pd_v[pl.ds(o, LANES)], off_d, scl_d)
            b = (z0 * GRID + y0) * GRID + x0
            for k in range(8):
                idx_v[pl.ds(g * 8 * LANES + k * LANES, LANES)] = b + _OFFS[k]

    def fire(idx_v, gat_v):
        pltpu.async_copy(dens_hbm.at[idx_v], gat_v, sem)

    def drain(idx_v, gat_v):
        pltpu.make_async_copy(dens_hbm.at[idx_v], gat_v, sem).wait()

    def blend_sub(si, gat_v, p):
        s0 = si * SUB
        for g in range(GROUPS):
            o = s0 + g * LANES
            _, fx = axis_prep(pw_v[pl.ds(o, LANES)], off_w, scl_w)
            _, fy = axis_prep(ph_v[pl.ds(o, LANES)], off_h, scl_h)
            _, fz = axis_prep(pd_v[pl.ds(o, LANES)], off_d, scl_d)
            gx = 1.0 - fx
            gy = 1.0 - fy
            gz = 1.0 - fz

            def corner(k):
                return gat_v[pl.ds(g * 8 * LANES + k * LANES, LANES)]

            c00 = corner(0) * gx + corner(1) * fx
            c01 = corner(2) * gx + corner(3) * fx
            c10 = corner(4) * gx + corner(5) * fx
            c11 = corner(6) * gx + corner(7) * fx
            c0 = c00 * gy + c01 * fy
            c1 = c10 * gy + c11 * fy
            d = c0 * gz + c1 * fz
            out_v[pl.ds(s0 + g * LANES, LANES)] = jnp.where(d >= thr, 1.0, 0.0)

    def chunk_body(ci, carry):
        base = base_w + ci * CHUNK
        pltpu.sync_copy(pw_hbm.at[pl.ds(base, CHUNK)], pw_v)
        pltpu.sync_copy(ph_hbm.at[pl.ds(base, CHUNK)], ph_v)
        pltpu.sync_copy(pd_hbm.at[pl.ds(base, CHUNK)], pd_v)

        bufs = ((idx_a, gat_a), (idx_b, gat_b), (idx_c, gat_c), (idx_d, gat_d))
        for si in range(3):
            compute_sub(si, bufs[si][0], si)
            fire(*bufs[si])

        def pipe_body(h, carry2):
            s0 = 4 * h + 3
            for j in range(4):
                si = s0 + j
                cb = (3 + j) % 4
                compute_sub(si, bufs[cb][0], cb)
                fire(*bufs[cb])
                drain(*bufs[j])
                blend_sub(si - 3, bufs[j][1], j)
            return carry2

        lax.fori_loop(0, (n_sub - 4) // 4, pipe_body, 0)
        si = n_sub - 1
        compute_sub(si, bufs[3][0], 3)
        fire(*bufs[3])
        for j in range(4):
            drain(*bufs[j])
            blend_sub(si - 3 + j, bufs[j][1], j)
        pltpu.sync_copy(out_v, out_hbm.at[pl.ds(base, CHUNK)])
        return carry

    lax.fori_loop(0, per_w // CHUNK, chunk_body, 0)


@functools.partial(jax.jit, static_argnums=(5,))
def _gridmask_sc(pw, ph, pd, dens_flat, params16, n_pts):
    kern = pl.kernel(
        functools.partial(_sc_body, n_pts),
        out_type=jax.ShapeDtypeStruct((n_pts,), jnp.float32),
        mesh=plsc.VectorSubcoreMesh(core_axis_name="c", subcore_axis_name="s",
                                    num_cores=2, num_subcores=16),
        scratch_types=[
            pltpu.VMEM((CHUNK,), jnp.float32),
            pltpu.VMEM((CHUNK,), jnp.float32),
            pltpu.VMEM((CHUNK,), jnp.float32),
            pltpu.VMEM((8 * SUB,), jnp.int32),
            pltpu.VMEM((8 * SUB,), jnp.int32),
            pltpu.VMEM((8 * SUB,), jnp.int32),
            pltpu.VMEM((8 * SUB,), jnp.int32),
            pltpu.VMEM((8 * SUB,), jnp.float32),
            pltpu.VMEM((8 * SUB,), jnp.float32),
            pltpu.VMEM((8 * SUB,), jnp.float32),
            pltpu.VMEM((8 * SUB,), jnp.float32),
            pltpu.VMEM((CHUNK,), jnp.float32),
            pltpu.VMEM((8, LANES), jnp.float32),
            pltpu.SemaphoreType.DMA,
        ],
    )
    return kern(pw, ph, pd, dens_flat, params16)


def kernel(xyz, density, xyz_min, xyz_max, act_shift, voxel_size_ratio,
           mask_cache_thres):
    shape = xyz.shape[:-1]
    pts = xyz.reshape(-1, 3)
    n_pts = pts.shape[0]
    pw = pts[:, 2]
    ph = pts[:, 1]
    pd = pts[:, 0]
    dens_flat = density.reshape(-1)

    c = -jnp.log1p(-mask_cache_thres) / voxel_size_ratio
    thr = jnp.log(jnp.expm1(c)) - act_shift

    scl = (GRID - 1.0) / (xyz_max - xyz_min)
    params = jnp.stack([
        xyz_min[2], scl[2],
        xyz_min[1], scl[1],
        xyz_min[0], scl[0],
        thr, jnp.float32(0.0),
    ]).astype(jnp.float32)
    params16 = jnp.broadcast_to(params[:, None], (8, LANES))

    outf = _gridmask_sc(pw, ph, pd, dens_flat, params16, n_pts)
    return (outf > 0.5).reshape(shape)


# speedup vs baseline: 8.9305x; 1.0211x over previous
import functools

import jax
import jax.numpy as jnp
from jax import lax
from jax.experimental import pallas as pl
from jax.experimental.pallas import tpu as pltpu
from jax.experimental.pallas import tpu_sc as plsc

GRID = 160
LANES = 16
SUB = 256
GROUPS = SUB // LANES
CHUNK = 8192
NW = 32

_OFFS = (0, 1, GRID, GRID + 1,
         GRID * GRID, GRID * GRID + 1, GRID * GRID + GRID, GRID * GRID + GRID + 1)


def _sc_body(n_pts, pw_hbm, ph_hbm, pd_hbm, dens_hbm, params_hbm, out_hbm,
             pw_v, ph_v, pd_v, idx_a, idx_b, idx_c, idx_d,
             gat_a, gat_b, gat_c, gat_d, out_v, params_v, sem):
    wid = lax.axis_index("s") * 2 + lax.axis_index("c")
    per_w = n_pts // NW
    base_w = wid * per_w
    n_sub = CHUNK // SUB

    pltpu.sync_copy(params_hbm, params_v)
    off_w = params_v[0]
    scl_w = params_v[1]
    off_h = params_v[2]
    scl_h = params_v[3]
    off_d = params_v[4]
    scl_d = params_v[5]
    thr = params_v[6]

    def axis_prep(vals, off, scl):
        s = jnp.maximum((vals - off) * scl, 0.0)
        i0 = jnp.minimum(s.astype(jnp.int32), GRID - 2)
        frac = s - i0.astype(jnp.float32)
        return i0, frac

    def compute_sub(si, idx_v, p):
        s0 = si * SUB
        for g in range(GROUPS):
            o = s0 + g * LANES
            x0, _ = axis_prep(pw_v[pl.ds(o, LANES)], off_w, scl_w)
            y0, _ = axis_prep(ph_v[pl.ds(o, LANES)], off_h, scl_h)
            z0, _ = axis_prep(pd_v[pl.ds(o, LANES)], off_d, scl_d)
            b = (z0 * GRID + y0) * GRID + x0
            for k in range(8):
                idx_v[pl.ds(g * 8 * LANES + k * LANES, LANES)] = b + _OFFS[k]

    def fire(idx_v, gat_v):
        pltpu.async_copy(dens_hbm.at[idx_v], gat_v, sem)

    def drain(idx_v, gat_v):
        pltpu.make_async_copy(dens_hbm.at[idx_v], gat_v, sem).wait()

    def blend_sub(si, gat_v, p):
        s0 = si * SUB
        for g in range(GROUPS):
            o = s0 + g * LANES
            _, fx = axis_prep(pw_v[pl.ds(o, LANES)], off_w, scl_w)
            _, fy = axis_prep(ph_v[pl.ds(o, LANES)], off_h, scl_h)
            _, fz = axis_prep(pd_v[pl.ds(o, LANES)], off_d, scl_d)
            gx = 1.0 - fx
            gy = 1.0 - fy
            gz = 1.0 - fz

            def corner(k):
                return gat_v[pl.ds(g * 8 * LANES + k * LANES, LANES)]

            c00 = corner(0) * gx + corner(1) * fx
            c01 = corner(2) * gx + corner(3) * fx
            c10 = corner(4) * gx + corner(5) * fx
            c11 = corner(6) * gx + corner(7) * fx
            c0 = c00 * gy + c01 * fy
            c1 = c10 * gy + c11 * fy
            d = c0 * gz + c1 * fz
            out_v[pl.ds(s0 + g * LANES, LANES)] = jnp.where(d >= thr, 1.0, 0.0)

    def chunk_body(ci, carry):
        base = base_w + ci * CHUNK
        pltpu.sync_copy(pw_hbm.at[pl.ds(base, CHUNK)], pw_v)
        pltpu.sync_copy(ph_hbm.at[pl.ds(base, CHUNK)], ph_v)
        pltpu.sync_copy(pd_hbm.at[pl.ds(base, CHUNK)], pd_v)

        bufs = ((idx_a, gat_a), (idx_b, gat_b), (idx_c, gat_c), (idx_d, gat_d))
        for si in range(3):
            compute_sub(si, bufs[si][0], si)
            fire(*bufs[si])

        def pipe_body(h, carry2):
            s0 = 4 * h + 3
            for j in range(4):
                si = s0 + j
                cb = (3 + j) % 4
                compute_sub(si, bufs[cb][0], cb)
                fire(*bufs[cb])
                drain(*bufs[j])
                blend_sub(si - 3, bufs[j][1], j)
            return carry2

        lax.fori_loop(0, (n_sub - 4) // 4, pipe_body, 0)
        si = n_sub - 1
        compute_sub(si, bufs[3][0], 3)
        fire(*bufs[3])
        for j in range(4):
            drain(*bufs[j])
            blend_sub(si - 3 + j, bufs[j][1], j)
        pltpu.sync_copy(out_v, out_hbm.at[pl.ds(base, CHUNK)])
        return carry

    lax.fori_loop(0, per_w // CHUNK, chunk_body, 0)


@functools.partial(jax.jit, static_argnums=(5,))
def _gridmask_sc(pw, ph, pd, dens_flat, params16, n_pts):
    kern = pl.kernel(
        functools.partial(_sc_body, n_pts),
        out_type=jax.ShapeDtypeStruct((n_pts,), jnp.float32),
        mesh=plsc.VectorSubcoreMesh(core_axis_name="c", subcore_axis_name="s",
                                    num_cores=2, num_subcores=16),
        scratch_types=[
            pltpu.VMEM((CHUNK,), jnp.float32),
            pltpu.VMEM((CHUNK,), jnp.float32),
            pltpu.VMEM((CHUNK,), jnp.float32),
            pltpu.VMEM((8 * SUB,), jnp.int32),
            pltpu.VMEM((8 * SUB,), jnp.int32),
            pltpu.VMEM((8 * SUB,), jnp.int32),
            pltpu.VMEM((8 * SUB,), jnp.int32),
            pltpu.VMEM((8 * SUB,), jnp.float32),
            pltpu.VMEM((8 * SUB,), jnp.float32),
            pltpu.VMEM((8 * SUB,), jnp.float32),
            pltpu.VMEM((8 * SUB,), jnp.float32),
            pltpu.VMEM((CHUNK,), jnp.float32),
            pltpu.VMEM((8, LANES), jnp.float32),
            pltpu.SemaphoreType.DMA,
        ],
    )
    return kern(pw, ph, pd, dens_flat, params16)


def kernel(xyz, density, xyz_min, xyz_max, act_shift, voxel_size_ratio,
           mask_cache_thres):
    shape = xyz.shape[:-1]
    pts = xyz.reshape(-1, 3)
    n_pts = pts.shape[0]
    pw = pts[:, 2]
    ph = pts[:, 1]
    pd = pts[:, 0]
    dens_flat = density.reshape(-1)

    c = -jnp.log1p(-mask_cache_thres) / voxel_size_ratio
    thr = jnp.log(jnp.expm1(c)) - act_shift

    scl = (GRID - 1.0) / (xyz_max - xyz_min)
    params = jnp.stack([
        xyz_min[2], scl[2],
        xyz_min[1], scl[1],
        xyz_min[0], scl[0],
        thr, jnp.float32(0.0),
    ]).astype(jnp.float32)
    params16 = jnp.broadcast_to(params[:, None], (8, LANES))

    outf = _gridmask_sc(pw, ph, pd, dens_flat, params16, n_pts)
    return (outf > 0.5).reshape(shape)


# speedup vs baseline: 9.0151x; 1.0095x over previous
import functools

import jax
import jax.numpy as jnp
from jax import lax
from jax.experimental import pallas as pl
from jax.experimental.pallas import tpu as pltpu
from jax.experimental.pallas import tpu_sc as plsc

GRID = 160
LANES = 16
SUB = 256
GROUPS = SUB // LANES
CHUNK = 16384
NW = 32

_OFFS = (0, 1, GRID, GRID + 1,
         GRID * GRID, GRID * GRID + 1, GRID * GRID + GRID, GRID * GRID + GRID + 1)


def _sc_body(n_pts, pw_hbm, ph_hbm, pd_hbm, dens_hbm, params_hbm, out_hbm,
             pw_v, ph_v, pd_v, idx_a, idx_b, idx_c, idx_d,
             gat_a, gat_b, gat_c, gat_d, out_v, params_v, sem):
    wid = lax.axis_index("s") * 2 + lax.axis_index("c")
    per_w = n_pts // NW
    base_w = wid * per_w
    n_sub = CHUNK // SUB

    pltpu.sync_copy(params_hbm, params_v)
    off_w = params_v[0]
    scl_w = params_v[1]
    off_h = params_v[2]
    scl_h = params_v[3]
    off_d = params_v[4]
    scl_d = params_v[5]
    thr = params_v[6]

    def axis_prep(vals, off, scl):
        s = jnp.maximum((vals - off) * scl, 0.0)
        i0 = jnp.minimum(s.astype(jnp.int32), GRID - 2)
        frac = s - i0.astype(jnp.float32)
        return i0, frac

    def compute_sub(si, idx_v, p):
        s0 = si * SUB
        for g in range(GROUPS):
            o = s0 + g * LANES
            x0, _ = axis_prep(pw_v[pl.ds(o, LANES)], off_w, scl_w)
            y0, _ = axis_prep(ph_v[pl.ds(o, LANES)], off_h, scl_h)
            z0, _ = axis_prep(pd_v[pl.ds(o, LANES)], off_d, scl_d)
            b = (z0 * GRID + y0) * GRID + x0
            for k in range(8):
                idx_v[pl.ds(g * 8 * LANES + k * LANES, LANES)] = b + _OFFS[k]

    def fire(idx_v, gat_v):
        pltpu.async_copy(dens_hbm.at[idx_v], gat_v, sem)

    def drain(idx_v, gat_v):
        pltpu.make_async_copy(dens_hbm.at[idx_v], gat_v, sem).wait()

    def blend_sub(si, gat_v, p):
        s0 = si * SUB
        for g in range(GROUPS):
            o = s0 + g * LANES
            _, fx = axis_prep(pw_v[pl.ds(o, LANES)], off_w, scl_w)
            _, fy = axis_prep(ph_v[pl.ds(o, LANES)], off_h, scl_h)
            _, fz = axis_prep(pd_v[pl.ds(o, LANES)], off_d, scl_d)
            gx = 1.0 - fx
            gy = 1.0 - fy
            gz = 1.0 - fz

            def corner(k):
                return gat_v[pl.ds(g * 8 * LANES + k * LANES, LANES)]

            c00 = corner(0) * gx + corner(1) * fx
            c01 = corner(2) * gx + corner(3) * fx
            c10 = corner(4) * gx + corner(5) * fx
            c11 = corner(6) * gx + corner(7) * fx
            c0 = c00 * gy + c01 * fy
            c1 = c10 * gy + c11 * fy
            d = c0 * gz + c1 * fz
            out_v[pl.ds(s0 + g * LANES, LANES)] = jnp.where(d >= thr, 1.0, 0.0)

    def chunk_body(ci, carry):
        base = base_w + ci * CHUNK
        pltpu.sync_copy(pw_hbm.at[pl.ds(base, CHUNK)], pw_v)
        pltpu.sync_copy(ph_hbm.at[pl.ds(base, CHUNK)], ph_v)
        pltpu.sync_copy(pd_hbm.at[pl.ds(base, CHUNK)], pd_v)

        bufs = ((idx_a, gat_a), (idx_b, gat_b), (idx_c, gat_c), (idx_d, gat_d))
        for si in range(3):
            compute_sub(si, bufs[si][0], si)
            fire(*bufs[si])

        def pipe_body(h, carry2):
            s0 = 4 * h + 3
            for j in range(4):
                si = s0 + j
                cb = (3 + j) % 4
                compute_sub(si, bufs[cb][0], cb)
                fire(*bufs[cb])
                drain(*bufs[j])
                blend_sub(si - 3, bufs[j][1], j)
            return carry2

        lax.fori_loop(0, (n_sub - 4) // 4, pipe_body, 0)
        si = n_sub - 1
        compute_sub(si, bufs[3][0], 3)
        fire(*bufs[3])
        for j in range(4):
            drain(*bufs[j])
            blend_sub(si - 3 + j, bufs[j][1], j)
        pltpu.sync_copy(out_v, out_hbm.at[pl.ds(base, CHUNK)])
        return carry

    lax.fori_loop(0, per_w // CHUNK, chunk_body, 0)


@functools.partial(jax.jit, static_argnums=(5,))
def _gridmask_sc(pw, ph, pd, dens_flat, params16, n_pts):
    kern = pl.kernel(
        functools.partial(_sc_body, n_pts),
        out_type=jax.ShapeDtypeStruct((n_pts,), jnp.float32),
        mesh=plsc.VectorSubcoreMesh(core_axis_name="c", subcore_axis_name="s",
                                    num_cores=2, num_subcores=16),
        scratch_types=[
            pltpu.VMEM((CHUNK,), jnp.float32),
            pltpu.VMEM((CHUNK,), jnp.float32),
            pltpu.VMEM((CHUNK,), jnp.float32),
            pltpu.VMEM((8 * SUB,), jnp.int32),
            pltpu.VMEM((8 * SUB,), jnp.int32),
            pltpu.VMEM((8 * SUB,), jnp.int32),
            pltpu.VMEM((8 * SUB,), jnp.int32),
            pltpu.VMEM((8 * SUB,), jnp.float32),
            pltpu.VMEM((8 * SUB,), jnp.float32),
            pltpu.VMEM((8 * SUB,), jnp.float32),
            pltpu.VMEM((8 * SUB,), jnp.float32),
            pltpu.VMEM((CHUNK,), jnp.float32),
            pltpu.VMEM((8, LANES), jnp.float32),
            pltpu.SemaphoreType.DMA,
        ],
    )
    return kern(pw, ph, pd, dens_flat, params16)


def kernel(xyz, density, xyz_min, xyz_max, act_shift, voxel_size_ratio,
           mask_cache_thres):
    shape = xyz.shape[:-1]
    pts = xyz.reshape(-1, 3)
    n_pts = pts.shape[0]
    pw = pts[:, 2]
    ph = pts[:, 1]
    pd = pts[:, 0]
    dens_flat = density.reshape(-1)

    c = -jnp.log1p(-mask_cache_thres) / voxel_size_ratio
    thr = jnp.log(jnp.expm1(c)) - act_shift

    scl = (GRID - 1.0) / (xyz_max - xyz_min)
    params = jnp.stack([
        xyz_min[2], scl[2],
        xyz_min[1], scl[1],
        xyz_min[0], scl[0],
        thr, jnp.float32(0.0),
    ]).astype(jnp.float32)
    params16 = jnp.broadcast_to(params[:, None], (8, LANES))

    outf = _gridmask_sc(pw, ph, pd, dens_flat, params16, n_pts)
    return (outf > 0.5).reshape(shape)
